# Initial kernel scaffold; baseline (speedup 1.0000x reference)
import jax
import jax.numpy as jnp
from jax.experimental import pallas as pl


def kernel(type_table, move_table, cond_table, movetype_idx, moveid_idx, condition_idx):
    raise NotImplementedError("write your pallas kernel here")


# speedup vs baseline: 3.7313x; 3.7313x over previous
import functools

import jax
import jax.numpy as jnp
from jax import lax
from jax.experimental import pallas as pl
from jax.experimental.pallas import tpu as pltpu
from jax.experimental.pallas import tpu_sc as plsc

_D = 128
_CH = 128
_NW = 32


def _phase(table, idx2d, out, idx_v, rows_v, sem, wid, nchunks):
    base = wid * nchunks * _CH
    pltpu.sync_copy(idx2d.at[wid], idx_v.at[pl.ds(0, nchunks)])

    def body(j, carry):
        pltpu.async_copy(table.at[idx_v.at[j]], rows_v, sem).wait()
        pltpu.sync_copy(rows_v, out.at[pl.ds(base + j * _CH, _CH)])
        return carry

    lax.fori_loop(0, nchunks, body, 0)


def _make_kernel(n_t, n_m, n_c):
    mesh = plsc.VectorSubcoreMesh(core_axis_name="c", subcore_axis_name="s")
    nmax = max(n_t, n_m, n_c)

    @functools.partial(
        pl.kernel,
        mesh=mesh,
        out_type=(
            jax.ShapeDtypeStruct((_NW * n_t * _CH, _D), jnp.float32),
            jax.ShapeDtypeStruct((_NW * n_m * _CH, _D), jnp.float32),
            jax.ShapeDtypeStruct((_NW * n_c * _CH, _D), jnp.float32),
        ),
        scratch_types=[
            pltpu.VMEM((nmax, _CH), jnp.int32),
            pltpu.VMEM((_CH, _D), jnp.float32),
            pltpu.SemaphoreType.DMA,
        ],
    )
    def k(type_table, move_table, cond_table, t_idx, m_idx, c_idx,
          t_out, m_out, c_out, idx_v, rows_v, sem):
        wid = lax.axis_index("s") * mesh.num_cores + lax.axis_index("c")
        _phase(type_table, t_idx, t_out, idx_v, rows_v, sem, wid, n_t)
        _phase(move_table, m_idx, m_out, idx_v, rows_v, sem, wid, n_m)
        _phase(cond_table, c_idx, c_out, idx_v, rows_v, sem, wid, n_c)

    return k


def kernel(type_table, move_table, cond_table, movetype_idx, moveid_idx,
           condition_idx):
    B, St = movetype_idx.shape
    _, Sm = moveid_idx.shape
    _, Sc = condition_idx.shape
    n_t = (B * St) // (_NW * _CH)
    n_m = (B * Sm) // (_NW * _CH)
    n_c = (B * Sc) // (_NW * _CH)

    t_idx = movetype_idx.reshape(_NW, n_t, _CH)
    m_idx = moveid_idx.reshape(_NW, n_m, _CH)
    c_idx = condition_idx.reshape(_NW, n_c, _CH)

    k = _make_kernel(n_t, n_m, n_c)
    t_out, m_out, c_out = k(type_table, move_table, cond_table,
                            t_idx, m_idx, c_idx)
    return (
        t_out.reshape(B, St, _D),
        m_out.reshape(B, Sm, _D),
        c_out.reshape(B, Sc, _D),
    )


# speedup vs baseline: 4.2157x; 1.1298x over previous
import functools

import jax
import jax.numpy as jnp
from jax import lax
from jax.experimental import pallas as pl
from jax.experimental.pallas import tpu as pltpu
from jax.experimental.pallas import tpu_sc as plsc

_D = 128
_CH = 128
_NW = 32


def _phase(table, idx2d, out, idx_v, buf0, buf1, gsem0, gsem1, wsem0, wsem1,
           wid, nchunks):
    base = wid * nchunks * _CH
    pltpu.sync_copy(idx2d.at[wid], idx_v.at[pl.ds(0, nchunks)])

    dummy_src = table.at[pl.ds(0, _CH)]

    pltpu.async_copy(table.at[idx_v.at[0]], buf0, gsem0)

    def proc(c, buf_c, gsem_c, buf_n, gsem_n, wsem_c, wsem_n):
        @pl.when(c > 0)
        def _():
            pltpu.make_async_copy(buf_n, out.at[pl.ds(base, _CH)],
                                  wsem_n).wait()

        @pl.when(c + 1 < nchunks)
        def _():
            pltpu.async_copy(table.at[idx_v.at[c + 1]], buf_n, gsem_n)

        pltpu.make_async_copy(dummy_src, buf_c, gsem_c).wait()
        pltpu.async_copy(buf_c, out.at[pl.ds(base + c * _CH, _CH)], wsem_c)

    def body(jj, carry):
        c = 2 * jj
        proc(c, buf0, gsem0, buf1, gsem1, wsem0, wsem1)
        proc(c + 1, buf1, gsem1, buf0, gsem0, wsem1, wsem0)
        return carry

    lax.fori_loop(0, nchunks // 2, body, 0)
    pltpu.make_async_copy(buf1, out.at[pl.ds(base, _CH)], wsem1).wait()


def _make_kernel(n_t, n_m, n_c):
    mesh = plsc.VectorSubcoreMesh(core_axis_name="c", subcore_axis_name="s")
    nmax = max(n_t, n_m, n_c)

    @functools.partial(
        pl.kernel,
        mesh=mesh,
        out_type=(
            jax.ShapeDtypeStruct((_NW * n_t * _CH, _D), jnp.float32),
            jax.ShapeDtypeStruct((_NW * n_m * _CH, _D), jnp.float32),
            jax.ShapeDtypeStruct((_NW * n_c * _CH, _D), jnp.float32),
        ),
        scratch_types=[
            pltpu.VMEM((nmax, _CH), jnp.int32),
            pltpu.VMEM((_CH, _D), jnp.float32),
            pltpu.VMEM((_CH, _D), jnp.float32),
            pltpu.SemaphoreType.DMA,
            pltpu.SemaphoreType.DMA,
            pltpu.SemaphoreType.DMA,
            pltpu.SemaphoreType.DMA,
        ],
    )
    def k(type_table, move_table, cond_table, t_idx, m_idx, c_idx,
          t_out, m_out, c_out, idx_v, buf0, buf1, gsem0, gsem1, wsem0, wsem1):
        wid = lax.axis_index("s") * mesh.num_cores + lax.axis_index("c")
        args = (idx_v, buf0, buf1, gsem0, gsem1, wsem0, wsem1, wid)
        _phase(type_table, t_idx, t_out, *args, n_t)
        _phase(move_table, m_idx, m_out, *args, n_m)
        _phase(cond_table, c_idx, c_out, *args, n_c)

    return k


def kernel(type_table, move_table, cond_table, movetype_idx, moveid_idx,
           condition_idx):
    B, St = movetype_idx.shape
    _, Sm = moveid_idx.shape
    _, Sc = condition_idx.shape
    n_t = (B * St) // (_NW * _CH)
    n_m = (B * Sm) // (_NW * _CH)
    n_c = (B * Sc) // (_NW * _CH)

    t_idx = movetype_idx.reshape(_NW, n_t, _CH)
    m_idx = moveid_idx.reshape(_NW, n_m, _CH)
    c_idx = condition_idx.reshape(_NW, n_c, _CH)

    k = _make_kernel(n_t, n_m, n_c)
    t_out, m_out, c_out = k(type_table, move_table, cond_table,
                            t_idx, m_idx, c_idx)
    return (
        t_out.reshape(B, St, _D),
        m_out.reshape(B, Sm, _D),
        c_out.reshape(B, Sc, _D),
    )


# speedup vs baseline: 4.2374x; 1.0051x over previous
import functools

import jax
import jax.numpy as jnp
from jax import lax
from jax.experimental import pallas as pl
from jax.experimental.pallas import tpu as pltpu
from jax.experimental.pallas import tpu_sc as plsc

_D = 128
_CH = 128
_NW = 32


_NBUF = 4


def _phase(table, idx2d, out, idx_v, bufs, gsems, wsems, wid, nchunks):
    base = wid * nchunks * _CH
    pltpu.sync_copy(idx2d.at[wid], idx_v.at[pl.ds(0, nchunks)])

    dummy_src = table.at[pl.ds(0, _CH)]

    for b in range(_NBUF - 1):
        pltpu.async_copy(table.at[idx_v.at[b]], bufs[b], gsems[b])

    def proc(c, b):
        bn = (b + _NBUF - 1) % _NBUF
        @pl.when(c > 0)
        def _():
            pltpu.make_async_copy(bufs[bn], out.at[pl.ds(base, _CH)],
                                  wsems[bn]).wait()

        @pl.when(c + _NBUF - 1 < nchunks)
        def _():
            pltpu.async_copy(table.at[idx_v.at[c + _NBUF - 1]],
                             bufs[bn], gsems[bn])

        pltpu.make_async_copy(dummy_src, bufs[b], gsems[b]).wait()
        pltpu.async_copy(bufs[b], out.at[pl.ds(base + c * _CH, _CH)],
                         wsems[b])

    def body(jj, carry):
        for b in range(_NBUF):
            proc(_NBUF * jj + b, b)
        return carry

    lax.fori_loop(0, nchunks // _NBUF, body, 0)
    pltpu.make_async_copy(bufs[_NBUF - 1], out.at[pl.ds(base, _CH)],
                          wsems[_NBUF - 1]).wait()


def _make_kernel(n_t, n_m, n_c):
    mesh = plsc.VectorSubcoreMesh(core_axis_name="c", subcore_axis_name="s")
    nmax = max(n_t, n_m, n_c)

    @functools.partial(
        pl.kernel,
        mesh=mesh,
        out_type=(
            jax.ShapeDtypeStruct((_NW * n_t * _CH, _D), jnp.float32),
            jax.ShapeDtypeStruct((_NW * n_m * _CH, _D), jnp.float32),
            jax.ShapeDtypeStruct((_NW * n_c * _CH, _D), jnp.float32),
        ),
        scratch_types=(
            [pltpu.VMEM((nmax, _CH), jnp.int32)]
            + [pltpu.VMEM((_CH, _D), jnp.float32)] * _NBUF
            + [pltpu.SemaphoreType.DMA] * (2 * _NBUF)
        ),
    )
    def k(type_table, move_table, cond_table, t_idx, m_idx, c_idx,
          t_out, m_out, c_out, idx_v, *rest):
        bufs = list(rest[:_NBUF])
        gsems = list(rest[_NBUF:2 * _NBUF])
        wsems = list(rest[2 * _NBUF:3 * _NBUF])
        wid = lax.axis_index("s") * mesh.num_cores + lax.axis_index("c")
        args = (idx_v, bufs, gsems, wsems, wid)
        _phase(type_table, t_idx, t_out, *args, n_t)
        _phase(move_table, m_idx, m_out, *args, n_m)
        _phase(cond_table, c_idx, c_out, *args, n_c)

    return k


def kernel(type_table, move_table, cond_table, movetype_idx, moveid_idx,
           condition_idx):
    B, St = movetype_idx.shape
    _, Sm = moveid_idx.shape
    _, Sc = condition_idx.shape
    n_t = (B * St) // (_NW * _CH)
    n_m = (B * Sm) // (_NW * _CH)
    n_c = (B * Sc) // (_NW * _CH)

    t_idx = movetype_idx.reshape(_NW, n_t, _CH)
    m_idx = moveid_idx.reshape(_NW, n_m, _CH)
    c_idx = condition_idx.reshape(_NW, n_c, _CH)

    k = _make_kernel(n_t, n_m, n_c)
    t_out, m_out, c_out = k(type_table, move_table, cond_table,
                            t_idx, m_idx, c_idx)
    return (
        t_out.reshape(B, St, _D),
        m_out.reshape(B, Sm, _D),
        c_out.reshape(B, Sc, _D),
    )


# speedup vs baseline: 5.1629x; 1.2184x over previous
import functools

import jax
import jax.numpy as jnp
from jax import lax
from jax.experimental import pallas as pl
from jax.experimental.pallas import tpu as pltpu
from jax.experimental.pallas import tpu_sc as plsc

_D = 128
_CH = 128
_NW = 32


_NBUF = 4


def _phase(table, idx2d, out, idx_v, bufs, gsems, wsems, wid, nchunks):
    base = wid * nchunks * _CH
    pltpu.sync_copy(idx2d.at[wid], idx_v.at[pl.ds(0, nchunks)])

    dummy_src = out.at[pl.ds(base, _CH)]

    for b in range(_NBUF - 1):
        pltpu.async_copy(table.at[idx_v.at[b]], bufs[b], gsems[b])

    def proc(c, b):
        bn = (b + _NBUF - 1) % _NBUF
        @pl.when(c > 0)
        def _():
            pltpu.make_async_copy(bufs[bn], out.at[pl.ds(base, _CH)],
                                  wsems[bn]).wait()

        @pl.when(c + _NBUF - 1 < nchunks)
        def _():
            pltpu.async_copy(table.at[idx_v.at[c + _NBUF - 1]],
                             bufs[bn], gsems[bn])

        pltpu.make_async_copy(dummy_src, bufs[b], gsems[b]).wait()
        pltpu.async_copy(bufs[b], out.at[pl.ds(base + c * _CH, _CH)],
                         wsems[b])

    def body(jj, carry):
        for b in range(_NBUF):
            proc(_NBUF * jj + b, b)
        return carry

    lax.fori_loop(0, nchunks // _NBUF, body, 0)
    pltpu.make_async_copy(bufs[_NBUF - 1], out.at[pl.ds(base, _CH)],
                          wsems[_NBUF - 1]).wait()


def _make_kernel(n_t, n_m, n_c):
    mesh = plsc.VectorSubcoreMesh(core_axis_name="c", subcore_axis_name="s")
    nmax = max(n_t, n_m, n_c)

    @functools.partial(
        pl.kernel,
        mesh=mesh,
        out_type=(
            jax.ShapeDtypeStruct((_NW * n_t * _CH, _D), jnp.float32),
            jax.ShapeDtypeStruct((_NW * n_m * _CH, _D), jnp.float32),
            jax.ShapeDtypeStruct((_NW * n_c * _CH, _D), jnp.float32),
        ),
        scratch_types=(
            [pltpu.VMEM((nmax, _CH), jnp.int32)]
            + [pltpu.VMEM((_CH, _D), jnp.float32)] * _NBUF
            + [pltpu.SemaphoreType.DMA] * (2 * _NBUF)
            + [pltpu.VMEM_SHARED((1000, _D), jnp.float32)] * 2
        ),
    )
    def k(type_table, move_table, cond_table, t_idx, m_idx, c_idx,
          t_out, m_out, c_out, idx_v, *rest):
        bufs = list(rest[:_NBUF])
        gsems = list(rest[_NBUF:2 * _NBUF])
        wsems = list(rest[2 * _NBUF:3 * _NBUF])
        type_sh, cond_sh = rest[3 * _NBUF:]
        wid = lax.axis_index("s") * mesh.num_cores + lax.axis_index("c")
        args = (idx_v, bufs, gsems, wsems, wid)

        @pl.when(lax.axis_index("s") == 0)
        def _():
            pltpu.sync_copy(type_table, type_sh)
            pltpu.sync_copy(cond_table, cond_sh)

        _phase(move_table, m_idx, m_out, *args, n_m)
        plsc.subcore_barrier()
        _phase(type_sh, t_idx, t_out, *args, n_t)
        _phase(cond_sh, c_idx, c_out, *args, n_c)

    return k


def kernel(type_table, move_table, cond_table, movetype_idx, moveid_idx,
           condition_idx):
    B, St = movetype_idx.shape
    _, Sm = moveid_idx.shape
    _, Sc = condition_idx.shape
    n_t = (B * St) // (_NW * _CH)
    n_m = (B * Sm) // (_NW * _CH)
    n_c = (B * Sc) // (_NW * _CH)

    t_idx = movetype_idx.reshape(_NW, n_t, _CH)
    m_idx = moveid_idx.reshape(_NW, n_m, _CH)
    c_idx = condition_idx.reshape(_NW, n_c, _CH)

    k = _make_kernel(n_t, n_m, n_c)
    t_out, m_out, c_out = k(type_table, move_table, cond_table,
                            t_idx, m_idx, c_idx)
    return (
        t_out.reshape(B, St, _D),
        m_out.reshape(B, Sm, _D),
        c_out.reshape(B, Sc, _D),
    )


# speedup vs baseline: 7.6300x; 1.4778x over previous
import functools

import jax
import jax.numpy as jnp
from jax import lax
from jax.experimental import pallas as pl
from jax.experimental.pallas import tpu as pltpu
from jax.experimental.pallas import tpu_sc as plsc

_D = 128
_NW = 32
_RPC = 96
_NBUF = 4


def _phase(table, idx1, out3, dummy, idx_v, bufs, gsems, wsems, wid, S,
           nspw):
    P = _RPC // S
    nchunks = nspw // P
    sbase = wid * nspw
    nrows = nspw * S

    pltpu.sync_copy(idx1.at[wid], idx_v.at[pl.ds(0, nrows)])

    def gather(c, b):
        pltpu.async_copy(table.at[idx_v.at[pl.ds(c * _RPC, _RPC)]],
                         bufs[b], gsems[b])

    for b in range(_NBUF - 1):
        gather(b, b)

    def proc(c, b):
        bn = (b + _NBUF - 1) % _NBUF
        @pl.when(c > 0)
        def _():
            pltpu.make_async_copy(bufs[bn], dummy, wsems[bn]).wait()

        @pl.when(c + _NBUF - 1 < nchunks)
        def _():
            gather(c + _NBUF - 1, bn)

        pltpu.make_async_copy(dummy, bufs[b], gsems[b]).wait()
        for s in range(P):
            pltpu.async_copy(bufs[b].at[pl.ds(s * S, S)],
                             out3.at[sbase + c * P + s], wsems[b])

    def body(jj, carry):
        for b in range(_NBUF):
            proc(_NBUF * jj + b, b)
        return carry

    lax.fori_loop(0, nchunks // _NBUF, body, 0)
    pltpu.make_async_copy(bufs[_NBUF - 1], dummy, wsems[_NBUF - 1]).wait()


def _make_kernel(B, St, Sm, Sc, Vt, Vc):
    mesh = plsc.VectorSubcoreMesh(core_axis_name="c", subcore_axis_name="s")
    nspw = B // _NW
    nmax = nspw * max(St, Sm, Sc)

    @functools.partial(
        pl.kernel,
        mesh=mesh,
        out_type=(
            jax.ShapeDtypeStruct((B, St, _D), jnp.float32),
            jax.ShapeDtypeStruct((B, Sm, _D), jnp.float32),
            jax.ShapeDtypeStruct((B, Sc, _D), jnp.float32),
        ),
        scratch_types=(
            [pltpu.VMEM((nmax,), jnp.int32)]
            + [pltpu.VMEM((_RPC, _D), jnp.float32)] * _NBUF
            + [pltpu.SemaphoreType.DMA] * (2 * _NBUF)
            + [pltpu.VMEM_SHARED((Vt, _D), jnp.float32),
               pltpu.VMEM_SHARED((Vc, _D), jnp.float32)]
        ),
        compiler_params=pltpu.CompilerParams(use_tc_tiling_on_sc=True),
    )
    def k(type_table, move_table, cond_table, t_idx, m_idx, c_idx,
          t_out, m_out, c_out, idx_v, *rest):
        bufs = list(rest[:_NBUF])
        gsems = list(rest[_NBUF:2 * _NBUF])
        wsems = list(rest[2 * _NBUF:3 * _NBUF])
        type_sh, cond_sh = rest[3 * _NBUF:]
        wid = lax.axis_index("s") * mesh.num_cores + lax.axis_index("c")
        dummy = move_table.at[pl.ds(0, _RPC)]
        args = (dummy, idx_v, bufs, gsems, wsems, wid)

        @pl.when(lax.axis_index("s") == 0)
        def _():
            pltpu.sync_copy(type_table, type_sh)
            pltpu.sync_copy(cond_table, cond_sh)

        _phase(move_table, m_idx, m_out, *args, Sm, nspw)
        plsc.subcore_barrier()
        _phase(type_sh, t_idx, t_out, *args, St, nspw)
        _phase(cond_sh, c_idx, c_out, *args, Sc, nspw)

    return k


def kernel(type_table, move_table, cond_table, movetype_idx, moveid_idx,
           condition_idx):
    B, St = movetype_idx.shape
    _, Sm = moveid_idx.shape
    _, Sc = condition_idx.shape

    t_idx = movetype_idx.reshape(_NW, -1)
    m_idx = moveid_idx.reshape(_NW, -1)
    c_idx = condition_idx.reshape(_NW, -1)

    k = _make_kernel(B, St, Sm, Sc, type_table.shape[0], cond_table.shape[0])
    return k(type_table, move_table, cond_table, t_idx, m_idx, c_idx)


# speedup vs baseline: 7.8900x; 1.0341x over previous
import functools

import jax
import jax.numpy as jnp
from jax import lax
from jax.experimental import pallas as pl
from jax.experimental.pallas import tpu as pltpu
from jax.experimental.pallas import tpu_sc as plsc

_D = 128
_NW = 32
_RPC = 96
_NBUF = 4

_MESH = plsc.VectorSubcoreMesh(core_axis_name="c", subcore_axis_name="s")


def _phase(table, idx1, out3, dummy, idx_v, bufs, gsems, wsems, wid, S,
           nspw):
    P = _RPC // S
    nchunks = nspw // P
    sbase = wid * nspw
    nrows = nspw * S

    pltpu.sync_copy(idx1.at[pl.ds(wid * nrows, nrows)],
                    idx_v.at[pl.ds(0, nrows)])

    def gather(c, b):
        pltpu.async_copy(table.at[idx_v.at[pl.ds(c * _RPC, _RPC)]],
                         bufs[b], gsems[b])

    for b in range(_NBUF - 1):
        gather(b, b)

    def proc(c, b):
        bn = (b + _NBUF - 1) % _NBUF
        @pl.when(c > 0)
        def _():
            pltpu.make_async_copy(bufs[bn], dummy, wsems[bn]).wait()

        @pl.when(c + _NBUF - 1 < nchunks)
        def _():
            gather(c + _NBUF - 1, bn)

        pltpu.make_async_copy(dummy, bufs[b], gsems[b]).wait()
        for s in range(P):
            pltpu.async_copy(bufs[b].at[pl.ds(s * S, S)],
                             out3.at[sbase + c * P + s], wsems[b])

    def body(jj, carry):
        for b in range(_NBUF):
            proc(_NBUF * jj + b, b)
        return carry

    lax.fori_loop(0, nchunks // _NBUF, body, 0)
    pltpu.make_async_copy(bufs[_NBUF - 1], dummy, wsems[_NBUF - 1]).wait()


def _scratch(nmax_rows):
    return (
        [pltpu.VMEM((nmax_rows,), jnp.int32)]
        + [pltpu.VMEM((_RPC, _D), jnp.float32)] * _NBUF
        + [pltpu.SemaphoreType.DMA] * (2 * _NBUF)
    )


def _split_scratch(rest):
    bufs = list(rest[:_NBUF])
    gsems = list(rest[_NBUF:2 * _NBUF])
    wsems = list(rest[2 * _NBUF:3 * _NBUF])
    return bufs, gsems, wsems, rest[3 * _NBUF:]


def _wid():
    return lax.axis_index("s") * _MESH.num_cores + lax.axis_index("c")


def _make_small_kernel(B, St, Sc, Vt, Vc):
    nspw = B // _NW
    nmax = nspw * max(St, Sc)

    @functools.partial(
        pl.kernel,
        mesh=_MESH,
        out_type=(
            jax.ShapeDtypeStruct((B, St, _D), jnp.float32),
            jax.ShapeDtypeStruct((B, Sc, _D), jnp.float32),
        ),
        scratch_types=(
            _scratch(nmax)
            + [pltpu.VMEM_SHARED((Vt, _D), jnp.float32),
               pltpu.VMEM_SHARED((Vc, _D), jnp.float32)]
        ),
        compiler_params=pltpu.CompilerParams(use_tc_tiling_on_sc=True),
    )
    def ka(type_table, cond_table, t_idx, c_idx, t_out, c_out, idx_v,
           *rest):
        bufs, gsems, wsems, (type_sh, cond_sh) = _split_scratch(rest)
        wid = _wid()
        dummy = type_table.at[pl.ds(0, _RPC)]
        args = (dummy, idx_v, bufs, gsems, wsems, wid)

        @pl.when(lax.axis_index("s") == 0)
        def _():
            pltpu.sync_copy(type_table, type_sh)
            pltpu.sync_copy(cond_table, cond_sh)

        plsc.subcore_barrier()
        _phase(type_sh, t_idx, t_out, *args, St, nspw)
        _phase(cond_sh, c_idx, c_out, *args, Sc, nspw)

    return ka


def _make_move_kernel(B, Sm):
    nspw = B // _NW

    @functools.partial(
        pl.kernel,
        mesh=_MESH,
        out_type=jax.ShapeDtypeStruct((B, Sm, _D), jnp.float32),
        scratch_types=_scratch(nspw * Sm),
        compiler_params=pltpu.CompilerParams(use_tc_tiling_on_sc=True),
    )
    def kb(move_table, m_idx, m_out, idx_v, *rest):
        bufs, gsems, wsems, _ = _split_scratch(rest)
        dummy = move_table.at[pl.ds(0, _RPC)]
        _phase(move_table, m_idx, m_out, dummy, idx_v, bufs, gsems, wsems,
               _wid(), Sm, nspw)

    return kb


def kernel(type_table, move_table, cond_table, movetype_idx, moveid_idx,
           condition_idx):
    B, St = movetype_idx.shape
    _, Sm = moveid_idx.shape
    _, Sc = condition_idx.shape

    ka = _make_small_kernel(B, St, Sc, type_table.shape[0],
                            cond_table.shape[0])
    kb = _make_move_kernel(B, Sm)
    t_out, c_out = ka(type_table, cond_table, movetype_idx.reshape(-1),
                      condition_idx.reshape(-1))
    m_out = kb(move_table, moveid_idx.reshape(-1))
    return (t_out, m_out, c_out)


# speedup vs baseline: 7.8958x; 1.0007x over previous
import functools

import jax
import jax.numpy as jnp
from jax import lax
from jax.experimental import pallas as pl
from jax.experimental.pallas import tpu as pltpu
from jax.experimental.pallas import tpu_sc as plsc

_D = 128
_NW = 32
_RPC = 96
_NBUF = 4

_MESH = plsc.VectorSubcoreMesh(core_axis_name="c", subcore_axis_name="s")


def _phase(table, idx1, out3, dummy, idx_v, bufs, gsems, wsems, wid, S,
           nspw):
    P = _RPC // S
    nchunks = nspw // P
    sbase = wid * nspw
    nrows = nspw * S

    pltpu.sync_copy(idx1.at[pl.ds(wid * nrows, nrows)],
                    idx_v.at[pl.ds(0, nrows)])

    def gather(c, b):
        pltpu.async_copy(table.at[idx_v.at[pl.ds(c * _RPC, _RPC)]],
                         bufs[b], gsems[b])

    for b in range(_NBUF - 1):
        gather(b, b)

    def proc(c, b):
        bn = (b + _NBUF - 1) % _NBUF
        @pl.when(c > 0)
        def _():
            pltpu.make_async_copy(bufs[bn], dummy, wsems[bn]).wait()

        @pl.when(c + _NBUF - 1 < nchunks)
        def _():
            gather(c + _NBUF - 1, bn)

        pltpu.make_async_copy(dummy, bufs[b], gsems[b]).wait()
        for s in range(P):
            pltpu.async_copy(bufs[b].at[pl.ds(s * S, S)],
                             out3.at[sbase + c * P + s], wsems[b])

    def body(jj, carry):
        for b in range(_NBUF):
            proc(_NBUF * jj + b, b)
        return carry

    lax.fori_loop(0, nchunks // _NBUF, body, 0)
    pltpu.make_async_copy(bufs[_NBUF - 1], dummy, wsems[_NBUF - 1]).wait()


def _scratch(nmax_rows):
    return (
        [pltpu.VMEM((nmax_rows,), jnp.int32)]
        + [pltpu.VMEM((_RPC, _D), jnp.float32)] * _NBUF
        + [pltpu.SemaphoreType.DMA] * (2 * _NBUF)
    )


def _split_scratch(rest):
    bufs = list(rest[:_NBUF])
    gsems = list(rest[_NBUF:2 * _NBUF])
    wsems = list(rest[2 * _NBUF:3 * _NBUF])
    return bufs, gsems, wsems, rest[3 * _NBUF:]


def _wid():
    return lax.axis_index("s") * _MESH.num_cores + lax.axis_index("c")


def _make_small_kernel(B, St, Sc, Vt, Vc):
    nspw = B // _NW
    nmax = nspw * max(St, Sc)

    @functools.partial(
        pl.kernel,
        mesh=_MESH,
        out_type=(
            jax.ShapeDtypeStruct((B, St, _D), jnp.float32),
            jax.ShapeDtypeStruct((B, Sc, _D), jnp.float32),
        ),
        scratch_types=(
            _scratch(nmax)
            + [pltpu.VMEM_SHARED((Vt, _D), jnp.float32),
               pltpu.VMEM_SHARED((Vc, _D), jnp.float32)]
        ),
        compiler_params=pltpu.CompilerParams(use_tc_tiling_on_sc=True, needs_layout_passes=True),
    )
    def ka(type_table, cond_table, t_idx, c_idx, t_out, c_out, idx_v,
           *rest):
        bufs, gsems, wsems, (type_sh, cond_sh) = _split_scratch(rest)
        wid = _wid()
        dummy = type_table.at[pl.ds(0, _RPC)]
        args = (dummy, idx_v, bufs, gsems, wsems, wid)

        @pl.when(lax.axis_index("s") == 0)
        def _():
            pltpu.sync_copy(type_table, type_sh)
            pltpu.sync_copy(cond_table, cond_sh)

        plsc.subcore_barrier()
        _phase(type_sh, t_idx, t_out, *args, St, nspw)
        _phase(cond_sh, c_idx, c_out, *args, Sc, nspw)

    return ka


def _make_move_kernel(B, Sm):
    nspw = B // _NW

    @functools.partial(
        pl.kernel,
        mesh=_MESH,
        out_type=jax.ShapeDtypeStruct((B, Sm, _D), jnp.float32),
        scratch_types=_scratch(nspw * Sm),
        compiler_params=pltpu.CompilerParams(use_tc_tiling_on_sc=True, needs_layout_passes=True),
    )
    def kb(move_table, m_idx, m_out, idx_v, *rest):
        bufs, gsems, wsems, _ = _split_scratch(rest)
        dummy = move_table.at[pl.ds(0, _RPC)]
        _phase(move_table, m_idx, m_out, dummy, idx_v, bufs, gsems, wsems,
               _wid(), Sm, nspw)

    return kb


def kernel(type_table, move_table, cond_table, movetype_idx, moveid_idx,
           condition_idx):
    B, St = movetype_idx.shape
    _, Sm = moveid_idx.shape
    _, Sc = condition_idx.shape

    ka = _make_small_kernel(B, St, Sc, type_table.shape[0],
                            cond_table.shape[0])
    kb = _make_move_kernel(B, Sm)
    t_out, c_out = ka(type_table, cond_table, movetype_idx.reshape(-1),
                      condition_idx.reshape(-1))
    m_out = kb(move_table, moveid_idx.reshape(-1))
    return (t_out, m_out, c_out)


# speedup vs baseline: 12.6945x; 1.6078x over previous
import functools

import jax
import jax.numpy as jnp
from jax import lax
from jax.experimental import pallas as pl
from jax.experimental.pallas import tpu as pltpu
from jax.experimental.pallas import tpu_sc as plsc

_D = 128
_NW = 32
_CH = 128
_NBUF = 4

_MESH = plsc.VectorSubcoreMesh(core_axis_name="c", subcore_axis_name="s")


def _phase(table, idx1, out, dummy, idx_v, bufs, gsems, wsems, wid, nrpw):
    nchunks = nrpw // _CH
    base = wid * nrpw

    pltpu.sync_copy(idx1.at[pl.ds(base, nrpw)], idx_v.at[pl.ds(0, nrpw)])

    def gather(c, b):
        pltpu.async_copy(table.at[idx_v.at[pl.ds(c * _CH, _CH)]],
                         bufs[b], gsems[b])

    for b in range(_NBUF - 1):
        gather(b, b)

    def proc(c, b):
        bn = (b + _NBUF - 1) % _NBUF
        @pl.when(c > 0)
        def _():
            pltpu.make_async_copy(bufs[bn], dummy, wsems[bn]).wait()

        @pl.when(c + _NBUF - 1 < nchunks)
        def _():
            gather(c + _NBUF - 1, bn)

        pltpu.make_async_copy(dummy, bufs[b], gsems[b]).wait()
        pltpu.async_copy(bufs[b], out.at[pl.ds(base + c * _CH, _CH)],
                         wsems[b])

    def body(jj, carry):
        for b in range(_NBUF):
            proc(_NBUF * jj + b, b)
        return carry

    lax.fori_loop(0, nchunks // _NBUF, body, 0)
    pltpu.make_async_copy(bufs[_NBUF - 1], dummy, wsems[_NBUF - 1]).wait()


def _make_kernel(B, St, Sm, Sc, Vt, Vc):
    nmax = B * Sm // _NW

    @functools.partial(
        pl.kernel,
        mesh=_MESH,
        out_type=(
            jax.ShapeDtypeStruct((B * St, _D), jnp.float32),
            jax.ShapeDtypeStruct((B * Sm, _D), jnp.float32),
            jax.ShapeDtypeStruct((B * Sc, _D), jnp.float32),
        ),
        scratch_types=(
            [pltpu.VMEM((nmax,), jnp.int32)]
            + [pltpu.VMEM((_CH, _D), jnp.float32)] * _NBUF
            + [pltpu.SemaphoreType.DMA] * (2 * _NBUF)
            + [pltpu.VMEM_SHARED((Vt, _D), jnp.float32),
               pltpu.VMEM_SHARED((Vc, _D), jnp.float32)]
        ),
    )
    def k(type_table, move_table, cond_table, t_idx, m_idx, c_idx,
          t_out, m_out, c_out, idx_v, *rest):
        bufs = list(rest[:_NBUF])
        gsems = list(rest[_NBUF:2 * _NBUF])
        wsems = list(rest[2 * _NBUF:3 * _NBUF])
        type_sh, cond_sh = rest[3 * _NBUF:]
        wid = lax.axis_index("s") * _MESH.num_cores + lax.axis_index("c")
        dummy = move_table.at[pl.ds(0, _CH)]
        args = (dummy, idx_v, bufs, gsems, wsems, wid)

        @pl.when(lax.axis_index("s") == 0)
        def _():
            pltpu.sync_copy(type_table, type_sh)
            pltpu.sync_copy(cond_table, cond_sh)

        _phase(move_table, m_idx, m_out, *args, B * Sm // _NW)
        plsc.subcore_barrier()
        _phase(type_sh, t_idx, t_out, *args, B * St // _NW)
        _phase(cond_sh, c_idx, c_out, *args, B * Sc // _NW)

    return k


def kernel(type_table, move_table, cond_table, movetype_idx, moveid_idx,
           condition_idx):
    B, St = movetype_idx.shape
    _, Sm = moveid_idx.shape
    _, Sc = condition_idx.shape

    k = _make_kernel(B, St, Sm, Sc, type_table.shape[0],
                     cond_table.shape[0])
    t_flat, m_flat, c_flat = k(
        type_table, move_table, cond_table,
        movetype_idx.T.reshape(-1),
        moveid_idx.reshape(-1),
        condition_idx.T.reshape(-1),
    )
    return (
        t_flat.reshape(St, B, _D).transpose(1, 0, 2),
        m_flat.reshape(B, Sm, _D),
        c_flat.reshape(Sc, B, _D).transpose(1, 0, 2),
    )
